# baseline (speedup 1.0000x reference)
import functools

import jax
import jax.numpy as jnp
from jax import lax
from jax.experimental import pallas as pl
from jax.experimental.pallas import tpu as pltpu
from jax.experimental.pallas import tpu_sc as plsc

_NF = 26
_V = 100001
_VP = 100096
_D = 32
_B = 1024
_L = 50
_NTOK = _B * _L
_NCORES = 2
_NSUB = 16
_NW = _NCORES * _NSUB
_NFLD = _NF + 1
_NU = _NFLD * _D
_UPW = _NU // _NW
_C = 6400
_NC = _NTOK // _C


def _make_kernel():
    mesh = plsc.VectorSubcoreMesh(core_axis_name="c", subcore_axis_name="s")

    @functools.partial(
        pl.kernel,
        out_type=jax.ShapeDtypeStruct((_NU, 1, _NTOK), jnp.float32),
        mesh=mesh,
        compiler_params=pltpu.CompilerParams(needs_layout_passes=False),
        scratch_types=[
            pltpu.VMEM((_VP,), jnp.float32),
            pltpu.VMEM((1, _C), jnp.int32),
            pltpu.VMEM((1, _C), jnp.int32),
            pltpu.VMEM((1, _C), jnp.float32),
            pltpu.VMEM((1, _C), jnp.float32),
            pltpu.SemaphoreType.DMA,
            pltpu.SemaphoreType.DMA,
        ],
    )
    def emb(flat, idx_all, out, slab_v, idx0, idx1, ov0, ov1, isem, osem):
        w = lax.axis_index("c") * _NSUB + lax.axis_index("s")
        iota16 = lax.iota(jnp.int32, 16)
        zero16 = iota16 * 0
        ibufs = (idx0, idx1)
        obufs = (ov0, ov1)

        def unit_body(j, carry):
            u = w * _UPW + j
            f = lax.shift_right_logical(u, 5)
            pltpu.sync_copy(flat.at[pl.ds(u * _VP, _VP)], slab_v)
            ih = [
                pltpu.async_copy(
                    idx_all.at[f, :, pl.ds(0, _C)], ibufs[0], isem
                )
            ]
            oh = []
            for c in range(_NC):
                bi = c % 2
                ih[c].wait()
                if c + 1 < _NC:
                    ih.append(
                        pltpu.async_copy(
                            idx_all.at[f, :, pl.ds((c + 1) * _C, _C)],
                            ibufs[1 - bi], isem,
                        )
                    )
                if c >= 2:
                    oh[c - 2].wait()
                idx_v = ibufs[bi]
                o_v = obufs[bi]

                def blk(b, carry2):
                    for k in range(8):
                        lane = b * 128 + k * 16 + iota16
                        iv = plsc.load_gather(idx_v, [zero16, lane])
                        vals = plsc.load_gather(slab_v, [iv])
                        plsc.store_scatter(o_v, [zero16, lane], vals)
                    return carry2

                lax.fori_loop(0, _C // 128, blk, 0)
                oh.append(
                    pltpu.async_copy(
                        o_v, out.at[u, :, pl.ds(c * _C, _C)], osem
                    )
                )
            oh[_NC - 2].wait()
            oh[_NC - 1].wait()
            return carry

        lax.fori_loop(0, _UPW, unit_body, 0)

    return emb


_EMB = _make_kernel()


def kernel(transactions_cat_features, product_feature, tables, product_table):
    trans = transactions_cat_features.astype(jnp.int32)
    idx_f = trans.reshape(_NF, _NTOK)
    idx_p = jnp.broadcast_to(
        product_feature.astype(jnp.int32)[:, None], (_B, _L)
    ).reshape(1, _NTOK)
    idx_all = jnp.concatenate([idx_f, idx_p], axis=0).reshape(_NFLD, 1, _NTOK)
    tpad = jnp.pad(
        jnp.transpose(tables, (0, 2, 1)), ((0, 0), (0, 0), (0, _VP - _V))
    ).reshape(_NF * _D * _VP)
    ppad = jnp.pad(
        jnp.transpose(product_table, (1, 0)), ((0, 0), (0, _VP - _V))
    ).reshape(_D * _VP)
    flat = jnp.concatenate([tpad, ppad])
    out_t = _EMB(flat, idx_all)
    eye = jnp.eye(_NU, dtype=jnp.float32)
    picked = jax.lax.dot_general(
        out_t.reshape(_NU, _NTOK), eye, (((0,), (0,)), ((), ())),
        preferred_element_type=jnp.float32,
        precision=jax.lax.Precision.HIGHEST,
    )
    return picked.reshape(_B, _L, _NU)


# speedup vs baseline: 1.1213x; 1.1213x over previous
import functools

import jax
import jax.numpy as jnp
from jax import lax
from jax.experimental import pallas as pl
from jax.experimental.pallas import tpu as pltpu
from jax.experimental.pallas import tpu_sc as plsc

_NF = 26
_V = 100001
_VP = 100096
_D = 32
_B = 1024
_L = 50
_NTOK = _B * _L
_NCORES = 2
_NSUB = 16
_NW = _NCORES * _NSUB
_NFLD = _NF + 1
_NU = _NFLD * _D
_UPW = _NU // _NW
_C = 6400
_NC = _NTOK // _C


def _make_kernel():
    mesh = plsc.VectorSubcoreMesh(core_axis_name="c", subcore_axis_name="s")

    @functools.partial(
        pl.kernel,
        out_type=jax.ShapeDtypeStruct((_NU, 1, _NTOK), jnp.float32),
        mesh=mesh,
        compiler_params=pltpu.CompilerParams(needs_layout_passes=False),
        scratch_types=[
            pltpu.VMEM((_VP,), jnp.float32),
            pltpu.VMEM((1, _C), jnp.int32),
            pltpu.VMEM((1, _C), jnp.int32),
            pltpu.VMEM((1, _C), jnp.float32),
            pltpu.VMEM((1, _C), jnp.float32),
            pltpu.SemaphoreType.DMA,
            pltpu.SemaphoreType.DMA,
        ],
    )
    def emb(flat, idx_all, out, slab_v, idx0, idx1, ov0, ov1, isem, osem):
        w = lax.axis_index("c") * _NSUB + lax.axis_index("s")
        iota16 = lax.iota(jnp.int32, 16)
        zero16 = iota16 * 0
        ibufs = (idx0, idx1)
        obufs = (ov0, ov1)

        def unit_body(j, carry):
            u = w * _UPW + j
            f = lax.shift_right_logical(u, 5)
            pltpu.sync_copy(flat.at[pl.ds(u * _VP, _VP)], slab_v)
            ih = [
                pltpu.async_copy(
                    idx_all.at[f, :, pl.ds(0, _C)], ibufs[0], isem
                )
            ]
            oh = []
            for c in range(_NC):
                bi = c % 2
                ih[c].wait()
                if c + 1 < _NC:
                    ih.append(
                        pltpu.async_copy(
                            idx_all.at[f, :, pl.ds((c + 1) * _C, _C)],
                            ibufs[1 - bi], isem,
                        )
                    )
                if c >= 2:
                    oh[c - 2].wait()
                idx_v = ibufs[bi]
                o_v = obufs[bi]

                def blk(b, carry2):
                    for k in range(8):
                        lane = b * 128 + k * 16 + iota16
                        iv = plsc.load_gather(idx_v, [zero16, lane])
                        vals = plsc.load_gather(slab_v, [iv])
                        plsc.store_scatter(o_v, [zero16, lane], vals)
                    return carry2

                lax.fori_loop(0, _C // 128, blk, 0)
                oh.append(
                    pltpu.async_copy(
                        o_v, out.at[u, :, pl.ds(c * _C, _C)], osem
                    )
                )
            oh[_NC - 2].wait()
            oh[_NC - 1].wait()
            return carry

        lax.fori_loop(0, _UPW, unit_body, 0)

    return emb


_EMB = _make_kernel()


def kernel(transactions_cat_features, product_feature, tables, product_table):
    trans = transactions_cat_features.astype(jnp.int32)
    idx_f = trans.reshape(_NF, _NTOK)
    idx_p = jnp.broadcast_to(
        product_feature.astype(jnp.int32)[:, None], (_B, _L)
    ).reshape(1, _NTOK)
    idx_all = jnp.concatenate([idx_f, idx_p], axis=0).reshape(_NFLD, 1, _NTOK)
    tpad = jnp.pad(
        jnp.transpose(tables, (0, 2, 1)), ((0, 0), (0, 0), (0, _VP - _V))
    ).reshape(_NF * _D * _VP)
    ppad = jnp.pad(
        jnp.transpose(product_table, (1, 0)), ((0, 0), (0, _VP - _V))
    ).reshape(_D * _VP)
    flat = jnp.concatenate([tpad, ppad])
    out_t = _EMB(flat, idx_all)
    eye = jnp.eye(_NU, dtype=jnp.float32)
    picked = jax.lax.dot_general(
        out_t.reshape(_NU, _NTOK), eye, (((0,), (0,)), ((), ())),
        preferred_element_type=jnp.float32,
    )
    return picked.reshape(_B, _L, _NU)
